# Initial kernel scaffold; baseline (speedup 1.0000x reference)
import jax
import jax.numpy as jnp
from jax.experimental import pallas as pl


def kernel(embeddings, labels, W, b):
    raise NotImplementedError("write your pallas kernel here")


# speedup vs baseline: 16.9806x; 16.9806x over previous
import jax
import jax.numpy as jnp
from jax.experimental import pallas as pl
from jax.experimental.pallas import tpu as pltpu

_B = 4096
_D = 1024
_K = 50
_P = 10
_SIGMA = 0.5
_ALPHA = 0.1

_RB = 256
_RBP = 512
_ITERS = 22


def _prep_body(emb_ref, w_ref, z_ref, sq_ref, zw_ref):
    x = emb_ref[...]
    ss = jnp.sum(x * x, axis=1, keepdims=True)
    norm = jnp.maximum(jnp.sqrt(ss), 1e-12)
    z = x / norm
    z_ref[...] = z
    sq_ref[...] = jnp.sum(z * z, axis=1)
    zw_ref[...] = jax.lax.dot_general(
        z, w_ref[...], (((1,), (0,)), ((), ())),
        preferred_element_type=jnp.float32)[:, 0]


def _knn_body(z_row_ref, z_all_ref, sq_row_ref, sq_all_ref, knn_ref, d2_ref):
    i = pl.program_id(0)
    g = jax.lax.dot_general(
        z_row_ref[...], z_all_ref[...], (((1,), (1,)), ((), ())),
        preferred_element_type=jnp.float32)
    sq_r = sq_row_ref[...][:, None]
    sq_c = sq_all_ref[...][None, :]
    d2 = sq_r + sq_c - 2.0 * g
    row_ids = i * _RB + jax.lax.broadcasted_iota(jnp.int32, (_RB, _B), 0)
    col_ids = jax.lax.broadcasted_iota(jnp.int32, (_RB, _B), 1)
    d2_ref[...] = jnp.where(row_ids == col_ids, 1e9, d2)

    def step(_, carry):
        lo, hi = carry
        mid = 0.5 * (lo + hi)
        cnt = jnp.sum((d2_ref[...] <= mid).astype(jnp.float32),
                      axis=1, keepdims=True)
        ge = cnt >= float(_K)
        return jnp.where(ge, lo, mid), jnp.where(ge, mid, hi)

    lo0 = jnp.zeros((_RB, 1), jnp.float32)
    hi0 = jnp.full((_RB, 1), 4.5, jnp.float32)
    _, hi = jax.lax.fori_loop(0, _ITERS, step, (lo0, hi0))
    knn_ref[...] = jnp.sqrt(hi[:, 0])


def _softplus(x):
    return jnp.maximum(x, 0.0) + jnp.log(1.0 + jnp.exp(-jnp.abs(x)))


def _final_body(knn_ref, zw_ref, noise_ref, w_ref, b_ref, out_ref):
    bval = b_ref[0]
    zw = zw_ref[...]
    gw = jax.lax.dot_general(
        noise_ref[...], w_ref[...], (((1,), (0,)), ((), ())),
        preferred_element_type=jnp.float32)
    id_loss = jnp.sum(_softplus(-(zw + bval))) / float(_B)
    iota = jax.lax.broadcasted_iota(jnp.int32, (1, _B), 1)
    v = knn_ref[...]
    ood_sum = jnp.float32(0.0)
    for p in range(_P):
        m = jnp.max(v)
        idx = jnp.min(jnp.where(v == m, iota, _B))
        hit = iota == idx
        zsel = jnp.sum(jnp.where(hit, zw, 0.0))
        ood_sum = ood_sum + _softplus(zsel + bval + _SIGMA * gw[p, 0])
        v = jnp.where(hit, -1.0, v)
    out = _ALPHA * (id_loss + ood_sum / float(_P))
    out_ref[...] = jnp.full((1, 1), out, jnp.float32)


def kernel(embeddings, labels, W, b):
    del labels
    emb = embeddings.astype(jnp.float32)
    w = W.astype(jnp.float32)

    z, sq, zw = pl.pallas_call(
        _prep_body,
        grid=(_B // _RBP,),
        in_specs=[
            pl.BlockSpec((_RBP, _D), lambda i: (i, 0)),
            pl.BlockSpec((_D, 1), lambda i: (0, 0)),
        ],
        out_specs=[
            pl.BlockSpec((_RBP, _D), lambda i: (i, 0)),
            pl.BlockSpec((_RBP,), lambda i: (i,)),
            pl.BlockSpec((_RBP,), lambda i: (i,)),
        ],
        out_shape=[
            jax.ShapeDtypeStruct((_B, _D), jnp.float32),
            jax.ShapeDtypeStruct((_B,), jnp.float32),
            jax.ShapeDtypeStruct((_B,), jnp.float32),
        ],
    )(emb, w)

    knn = pl.pallas_call(
        _knn_body,
        grid=(_B // _RB,),
        in_specs=[
            pl.BlockSpec((_RB, _D), lambda i: (i, 0)),
            pl.BlockSpec((_B, _D), lambda i: (0, 0)),
            pl.BlockSpec((_RB,), lambda i: (i,)),
            pl.BlockSpec((_B,), lambda i: (0,)),
        ],
        out_specs=pl.BlockSpec((_RB,), lambda i: (i,)),
        out_shape=jax.ShapeDtypeStruct((_B,), jnp.float32),
        scratch_shapes=[pltpu.VMEM((_RB, _B), jnp.float32)],
    )(z, z, sq, sq)

    noise = jax.random.normal(jax.random.key(1234), (_P, 1, _D),
                              dtype=jnp.float32).reshape(_P, _D)
    out = pl.pallas_call(
        _final_body,
        in_specs=[
            pl.BlockSpec((1, _B), lambda: (0, 0)),
            pl.BlockSpec((1, _B), lambda: (0, 0)),
            pl.BlockSpec((_P, _D), lambda: (0, 0)),
            pl.BlockSpec((_D, 1), lambda: (0, 0)),
            pl.BlockSpec(memory_space=pltpu.SMEM),
        ],
        out_specs=pl.BlockSpec((1, 1), lambda: (0, 0)),
        out_shape=jax.ShapeDtypeStruct((1, 1), jnp.float32),
    )(knn.reshape(1, _B), zw.reshape(1, _B), noise, w,
      b.astype(jnp.float32))
    return out.reshape(())
